# Initial kernel scaffold; baseline (speedup 1.0000x reference)
import jax
import jax.numpy as jnp
from jax.experimental import pallas as pl


def kernel(embedding, kgg_table, rel_table, scg_ids, relation_ids, kgg_ids):
    raise NotImplementedError("write your pallas kernel here")


# speedup vs baseline: 2.0472x; 2.0472x over previous
import functools

import jax
import jax.numpy as jnp
from jax import lax
from jax.experimental import pallas as pl
from jax.experimental.pallas import tpu as pltpu
from jax.experimental.pallas import tpu_sc as plsc

B = 16384
D = 128
NC = 2
NS = 16
NW = NC * NS
B_PER_W = B // NW
CHUNK = 128
N_CHUNKS = B_PER_W // CHUNK


def _gather3_body(emb_hbm, kgg_hbm, rel_hbm, scg_i_hbm, rel_i_hbm,
                  kgg_i_hbm, scg_out, kgg_out, rel_out,
                  idx_v, rows_v, sem):
    wid = lax.axis_index("s") * NC + lax.axis_index("c")
    base = wid * B_PER_W
    jobs = (
        (emb_hbm, scg_i_hbm, scg_out),
        (kgg_hbm, kgg_i_hbm, kgg_out),
        (rel_hbm, rel_i_hbm, rel_out),
    )
    for tab, ids, out in jobs:
        for j in range(N_CHUNKS):
            off = base + j * CHUNK
            pltpu.sync_copy(ids.at[pl.ds(off, CHUNK)], idx_v)
            pltpu.async_copy(tab.at[idx_v], rows_v, sem).wait()
            pltpu.sync_copy(rows_v, out.at[pl.ds(off, CHUNK)])


@jax.jit
def _gather3(embedding, kgg_table, rel_table, scg_ids, relation_ids, kgg_ids):
    mesh = plsc.VectorSubcoreMesh(core_axis_name="c", subcore_axis_name="s")
    f = functools.partial(
        pl.kernel,
        mesh=mesh,
        out_type=(
            jax.ShapeDtypeStruct((B, D), jnp.float32),
            jax.ShapeDtypeStruct((B, D), jnp.float32),
            jax.ShapeDtypeStruct((B, D), jnp.float32),
        ),
        scratch_types=[
            pltpu.VMEM((CHUNK,), jnp.int32),
            pltpu.VMEM((CHUNK, D), jnp.float32),
            pltpu.SemaphoreType.DMA,
        ],
    )(_gather3_body)
    return f(embedding, kgg_table, rel_table, scg_ids, relation_ids, kgg_ids)


def kernel(embedding, kgg_table, rel_table, scg_ids, relation_ids, kgg_ids):
    scg_ids = scg_ids.astype(jnp.int32)
    relation_ids = relation_ids.astype(jnp.int32)
    kgg_ids = kgg_ids.astype(jnp.int32)
    return _gather3(embedding, kgg_table, rel_table,
                    scg_ids, relation_ids, kgg_ids)


# speedup vs baseline: 2.5257x; 1.2337x over previous
import functools

import jax
import jax.numpy as jnp
from jax import lax
from jax.experimental import pallas as pl
from jax.experimental.pallas import tpu as pltpu
from jax.experimental.pallas import tpu_sc as plsc

B = 16384
D = 128
NC = 2
NS = 16
NW = NC * NS
B_PER_W = B // NW
CHUNK = 128
N_CHUNKS = B_PER_W // CHUNK


N_JOBS = 3 * N_CHUNKS
NB = 4


def _gather3_body(emb_hbm, kgg_hbm, rel_hbm, scg_i_hbm, rel_i_hbm,
                  kgg_i_hbm, scg_out, kgg_out, rel_out,
                  idx_v, *rest):
    rows = rest[:NB]
    isem = rest[NB]
    gsems = rest[NB + 1:2 * NB + 1]
    ssems = rest[2 * NB + 1:]
    wid = lax.axis_index("s") * NC + lax.axis_index("c")
    base = wid * B_PER_W
    tables = (
        (emb_hbm, scg_i_hbm, scg_out),
        (kgg_hbm, kgg_i_hbm, kgg_out),
        (rel_hbm, rel_i_hbm, rel_out),
    )
    jobs = []
    for tab, ids, out in tables:
        for j in range(N_CHUNKS):
            jobs.append((tab, ids, out, base + j * CHUNK))

    idx_copies = []
    for i, (tab, ids, out, off) in enumerate(jobs):
        idx_copies.append(
            pltpu.async_copy(ids.at[pl.ds(off, CHUNK)], idx_v.at[i], isem))
    for c in idx_copies:
        c.wait()

    gather_copies = [None] * N_JOBS
    store_copies = [None] * N_JOBS

    def fire_gather(i):
        tab, ids, out, off = jobs[i]
        gather_copies[i] = pltpu.async_copy(
            tab.at[idx_v.at[i]], rows[i % NB], gsems[i % NB])

    for i in range(NB):
        fire_gather(i)
    for i in range(N_JOBS):
        tab, ids, out, off = jobs[i]
        gather_copies[i].wait()
        store_copies[i] = pltpu.async_copy(
            rows[i % NB], out.at[pl.ds(off, CHUNK)], ssems[i % NB])
        nxt = i + NB
        if nxt < N_JOBS:
            store_copies[i].wait()
            fire_gather(nxt)
    for i in range(N_JOBS - NB, N_JOBS):
        store_copies[i].wait()


@jax.jit
def _gather3(embedding, kgg_table, rel_table, scg_ids, relation_ids, kgg_ids):
    mesh = plsc.VectorSubcoreMesh(core_axis_name="c", subcore_axis_name="s")
    f = functools.partial(
        pl.kernel,
        mesh=mesh,
        out_type=(
            jax.ShapeDtypeStruct((B, D), jnp.float32),
            jax.ShapeDtypeStruct((B, D), jnp.float32),
            jax.ShapeDtypeStruct((B, D), jnp.float32),
        ),
        scratch_types=(
            [pltpu.VMEM((N_JOBS, CHUNK), jnp.int32)]
            + [pltpu.VMEM((CHUNK, D), jnp.float32) for _ in range(NB)]
            + [pltpu.SemaphoreType.DMA for _ in range(2 * NB + 1)]
        ),
    )(_gather3_body)
    return f(embedding, kgg_table, rel_table, scg_ids, relation_ids, kgg_ids)


def kernel(embedding, kgg_table, rel_table, scg_ids, relation_ids, kgg_ids):
    scg_ids = scg_ids.astype(jnp.int32)
    relation_ids = relation_ids.astype(jnp.int32)
    kgg_ids = kgg_ids.astype(jnp.int32)
    return _gather3(embedding, kgg_table, rel_table,
                    scg_ids, relation_ids, kgg_ids)


# speedup vs baseline: 2.5503x; 1.0098x over previous
import functools

import jax
import jax.numpy as jnp
from jax import lax
from jax.experimental import pallas as pl
from jax.experimental.pallas import tpu as pltpu
from jax.experimental.pallas import tpu_sc as plsc

B = 16384
D = 128
NC = 2
NS = 16
NW = NC * NS
B_PER_W = B // NW
CHUNK = 128
N_CHUNKS = B_PER_W // CHUNK


N_JOBS = 3 * N_CHUNKS
NB = 6


def _gather3_body(emb_hbm, kgg_hbm, rel_hbm, scg_i_hbm, rel_i_hbm,
                  kgg_i_hbm, scg_out, kgg_out, rel_out,
                  idx_v, *rest):
    rows = rest[:NB]
    isem = rest[NB]
    gsems = rest[NB + 1:2 * NB + 1]
    ssems = rest[2 * NB + 1:]
    wid = lax.axis_index("s") * NC + lax.axis_index("c")
    base = wid * B_PER_W
    tables = (
        (emb_hbm, scg_i_hbm, scg_out),
        (kgg_hbm, kgg_i_hbm, kgg_out),
        (rel_hbm, rel_i_hbm, rel_out),
    )
    jobs = []
    for tab, ids, out in tables:
        for j in range(N_CHUNKS):
            jobs.append((tab, ids, out, base + j * CHUNK))

    idx_copies = []
    for i, (tab, ids, out, off) in enumerate(jobs):
        idx_copies.append(
            pltpu.async_copy(ids.at[pl.ds(off, CHUNK)], idx_v.at[i], isem))
    for c in idx_copies:
        c.wait()

    gather_copies = [None] * N_JOBS
    store_copies = [None] * N_JOBS

    def fire_gather(i):
        tab, ids, out, off = jobs[i]
        gather_copies[i] = pltpu.async_copy(
            tab.at[idx_v.at[i]], rows[i % NB], gsems[i % NB])

    for i in range(NB):
        fire_gather(i)
    for i in range(N_JOBS):
        tab, ids, out, off = jobs[i]
        gather_copies[i].wait()
        store_copies[i] = pltpu.async_copy(
            rows[i % NB], out.at[pl.ds(off, CHUNK)], ssems[i % NB])
        nxt = i + NB
        if nxt < N_JOBS:
            store_copies[i].wait()
            fire_gather(nxt)
    for i in range(N_JOBS - NB, N_JOBS):
        store_copies[i].wait()


@jax.jit
def _gather3(embedding, kgg_table, rel_table, scg_ids, relation_ids, kgg_ids):
    mesh = plsc.VectorSubcoreMesh(core_axis_name="c", subcore_axis_name="s")
    f = functools.partial(
        pl.kernel,
        mesh=mesh,
        out_type=(
            jax.ShapeDtypeStruct((B, D), jnp.float32),
            jax.ShapeDtypeStruct((B, D), jnp.float32),
            jax.ShapeDtypeStruct((B, D), jnp.float32),
        ),
        scratch_types=(
            [pltpu.VMEM((N_JOBS, CHUNK), jnp.int32)]
            + [pltpu.VMEM((CHUNK, D), jnp.float32) for _ in range(NB)]
            + [pltpu.SemaphoreType.DMA for _ in range(2 * NB + 1)]
        ),
    )(_gather3_body)
    return f(embedding, kgg_table, rel_table, scg_ids, relation_ids, kgg_ids)


def kernel(embedding, kgg_table, rel_table, scg_ids, relation_ids, kgg_ids):
    scg_ids = scg_ids.astype(jnp.int32)
    relation_ids = relation_ids.astype(jnp.int32)
    kgg_ids = kgg_ids.astype(jnp.int32)
    return _gather3(embedding, kgg_table, rel_table,
                    scg_ids, relation_ids, kgg_ids)


# speedup vs baseline: 2.5537x; 1.0013x over previous
import functools

import jax
import jax.numpy as jnp
from jax import lax
from jax.experimental import pallas as pl
from jax.experimental.pallas import tpu as pltpu
from jax.experimental.pallas import tpu_sc as plsc

B = 16384
D = 128
NC = 2
NS = 16
NW = NC * NS
B_PER_W = B // NW
CHUNK = 128
N_CHUNKS = B_PER_W // CHUNK


N_JOBS = 3 * N_CHUNKS
NB = 6


def _gather3_body(emb_hbm, kgg_hbm, rel_hbm, scg_i_hbm, rel_i_hbm,
                  kgg_i_hbm, scg_out, kgg_out, rel_out,
                  idx_v, *rest):
    rows = rest[:NB]
    isem = rest[NB]
    gsems = rest[NB + 1:2 * NB + 1]
    ssems = rest[2 * NB + 1:]
    wid = lax.axis_index("s") * NC + lax.axis_index("c")
    base = wid * B_PER_W
    tables = (
        (emb_hbm, scg_i_hbm, scg_out),
        (kgg_hbm, kgg_i_hbm, kgg_out),
        (rel_hbm, rel_i_hbm, rel_out),
    )
    jobs = []
    for tab, ids2d, out in tables:
        for j in range(N_CHUNKS):
            jobs.append((tab, out, base + j * CHUNK))

    idx_copies = []
    for t, (tab, ids2d, out) in enumerate(tables):
        idx_copies.append(pltpu.async_copy(
            ids2d.at[pl.ds(wid * N_CHUNKS, N_CHUNKS)],
            idx_v.at[pl.ds(t * N_CHUNKS, N_CHUNKS)], isem))
    for c in idx_copies:
        c.wait()

    gather_copies = [None] * N_JOBS
    store_copies = [None] * N_JOBS

    def fire_gather(i):
        tab, out, off = jobs[i]
        gather_copies[i] = pltpu.async_copy(
            tab.at[idx_v.at[i]], rows[i % NB], gsems[i % NB])

    def fire_store(i):
        tab, out, off = jobs[i]
        gather_copies[i].wait()
        store_copies[i] = pltpu.async_copy(
            rows[i % NB], out.at[pl.ds(off, CHUNK)], ssems[i % NB])

    for i in range(NB):
        fire_gather(i)
    for i in range(NB):
        fire_store(i)
    for i in range(NB, N_JOBS):
        store_copies[i - NB].wait()
        fire_gather(i)
    for i in range(NB, N_JOBS):
        fire_store(i)
    for i in range(N_JOBS - NB, N_JOBS):
        store_copies[i].wait()


@jax.jit
def _gather3(embedding, kgg_table, rel_table, scg_ids, relation_ids, kgg_ids):
    mesh = plsc.VectorSubcoreMesh(core_axis_name="c", subcore_axis_name="s")
    f = functools.partial(
        pl.kernel,
        mesh=mesh,
        out_type=(
            jax.ShapeDtypeStruct((B, D), jnp.float32),
            jax.ShapeDtypeStruct((B, D), jnp.float32),
            jax.ShapeDtypeStruct((B, D), jnp.float32),
        ),
        scratch_types=(
            [pltpu.VMEM((N_JOBS, CHUNK), jnp.int32)]
            + [pltpu.VMEM((CHUNK, D), jnp.float32) for _ in range(NB)]
            + [pltpu.SemaphoreType.DMA for _ in range(2 * NB + 1)]
        ),
    )(_gather3_body)
    return f(embedding, kgg_table, rel_table,
             scg_ids.reshape(B // CHUNK, CHUNK),
             relation_ids.reshape(B // CHUNK, CHUNK),
             kgg_ids.reshape(B // CHUNK, CHUNK))


def kernel(embedding, kgg_table, rel_table, scg_ids, relation_ids, kgg_ids):
    scg_ids = scg_ids.astype(jnp.int32)
    relation_ids = relation_ids.astype(jnp.int32)
    kgg_ids = kgg_ids.astype(jnp.int32)
    return _gather3(embedding, kgg_table, rel_table,
                    scg_ids, relation_ids, kgg_ids)


# speedup vs baseline: 2.5955x; 1.0164x over previous
import functools

import jax
import jax.numpy as jnp
from jax import lax
from jax.experimental import pallas as pl
from jax.experimental.pallas import tpu as pltpu
from jax.experimental.pallas import tpu_sc as plsc

B = 16384
D = 128
NC = 2
NS = 16
NW = NC * NS
B_PER_W = B // NW
CHUNK = 128
N_CHUNKS = B_PER_W // CHUNK


N_JOBS = 3 * N_CHUNKS
GPS = 2
N_SJ = N_JOBS // GPS
NB = 3


def _gather3_body(emb_hbm, kgg_hbm, rel_hbm, scg_i_hbm, rel_i_hbm,
                  kgg_i_hbm, scg_out, kgg_out, rel_out,
                  idx_v, *rest):
    rows = rest[:NB]
    isem = rest[NB]
    gsems = rest[NB + 1:2 * NB + 1]
    ssems = rest[2 * NB + 1:]
    wid = lax.axis_index("s") * NC + lax.axis_index("c")
    base = wid * B_PER_W
    tables = (
        (emb_hbm, scg_i_hbm, scg_out),
        (kgg_hbm, kgg_i_hbm, kgg_out),
        (rel_hbm, rel_i_hbm, rel_out),
    )
    jobs = []
    for tab, ids2d, out in tables:
        for j in range(N_CHUNKS):
            jobs.append((tab, out, base + j * CHUNK))

    idx_copies = []
    for t, (tab, ids2d, out) in enumerate(tables):
        idx_copies.append(pltpu.async_copy(
            ids2d.at[pl.ds(wid * N_CHUNKS, N_CHUNKS)],
            idx_v.at[pl.ds(t * N_CHUNKS, N_CHUNKS)], isem))
    for c in idx_copies:
        c.wait()

    gather_copies = [None] * N_JOBS
    store_copies = [None] * N_SJ

    def fire_gathers(sj):
        b = sj % NB
        for g in range(GPS):
            i = sj * GPS + g
            tab, out, off = jobs[i]
            gather_copies[i] = pltpu.async_copy(
                tab.at[idx_v.at[i]], rows[b].at[pl.ds(g * CHUNK, CHUNK)],
                gsems[b])

    def fire_store(sj):
        b = sj % NB
        tab, out, off = jobs[sj * GPS]
        for g in range(GPS):
            gather_copies[sj * GPS + g].wait()
        store_copies[sj] = pltpu.async_copy(
            rows[b], out.at[pl.ds(off, GPS * CHUNK)], ssems[b])

    for sj in range(NB):
        fire_gathers(sj)
    for sj in range(NB):
        fire_store(sj)
    for sj in range(NB, N_SJ):
        store_copies[sj - NB].wait()
        fire_gathers(sj)
    for sj in range(NB, N_SJ):
        fire_store(sj)
    for sj in range(N_SJ - NB, N_SJ):
        store_copies[sj].wait()


@jax.jit
def _gather3(embedding, kgg_table, rel_table, scg_ids, relation_ids, kgg_ids):
    mesh = plsc.VectorSubcoreMesh(core_axis_name="c", subcore_axis_name="s")
    f = functools.partial(
        pl.kernel,
        mesh=mesh,
        out_type=(
            jax.ShapeDtypeStruct((B, D), jnp.float32),
            jax.ShapeDtypeStruct((B, D), jnp.float32),
            jax.ShapeDtypeStruct((B, D), jnp.float32),
        ),
        scratch_types=(
            [pltpu.VMEM((N_JOBS, CHUNK), jnp.int32)]
            + [pltpu.VMEM((GPS * CHUNK, D), jnp.float32) for _ in range(NB)]
            + [pltpu.SemaphoreType.DMA for _ in range(2 * NB + 1)]
        ),
    )(_gather3_body)
    return f(embedding, kgg_table, rel_table,
             scg_ids.reshape(B // CHUNK, CHUNK),
             relation_ids.reshape(B // CHUNK, CHUNK),
             kgg_ids.reshape(B // CHUNK, CHUNK))


def kernel(embedding, kgg_table, rel_table, scg_ids, relation_ids, kgg_ids):
    scg_ids = scg_ids.astype(jnp.int32)
    relation_ids = relation_ids.astype(jnp.int32)
    kgg_ids = kgg_ids.astype(jnp.int32)
    return _gather3(embedding, kgg_table, rel_table,
                    scg_ids, relation_ids, kgg_ids)


# speedup vs baseline: 3.0096x; 1.1595x over previous
import functools

import jax
import jax.numpy as jnp
from jax import lax
from jax.experimental import pallas as pl
from jax.experimental.pallas import tpu as pltpu
from jax.experimental.pallas import tpu_sc as plsc

B = 16384
D = 128
NC = 2
NS = 16
NW = NC * NS
B_PER_W = B // NW
CHUNK = 128
N_CHUNKS = B_PER_W // CHUNK
N_JOBS = 3 * N_CHUNKS
GPS = 2
N_SJ = N_JOBS // GPS
NB = 3
REL_PAD = 1024
REL_PER_TILE = REL_PAD // NS


def _gather3_body(emb_hbm, kgg_hbm, rel_hbm, scg_i_hbm, rel_i_hbm,
                  kgg_i_hbm, scg_out, kgg_out, rel_out,
                  idx_v, stage_v, rel_sh, *rest):
    rows = rest[:NB]
    isem = rest[NB]
    stsem = rest[NB + 1]
    gsems = rest[NB + 2:2 * NB + 2]
    ssems = rest[2 * NB + 2:]
    cid = lax.axis_index("c")
    sid = lax.axis_index("s")
    wid = sid * NC + cid
    base = wid * B_PER_W
    tables = (
        (emb_hbm, scg_i_hbm, scg_out),
        (kgg_hbm, kgg_i_hbm, kgg_out),
        (rel_sh, rel_i_hbm, rel_out),
    )
    jobs = []
    for tab, ids2d, out in tables:
        for j in range(N_CHUNKS):
            jobs.append((tab, out, base + j * CHUNK))

    stage_in = pltpu.async_copy(
        rel_hbm.at[pl.ds(sid * REL_PER_TILE, REL_PER_TILE)], stage_v, stsem)
    idx_copies = []
    for t, (tab, ids2d, out) in enumerate(tables):
        idx_copies.append(pltpu.async_copy(
            ids2d.at[pl.ds(wid * N_CHUNKS, N_CHUNKS)],
            idx_v.at[pl.ds(t * N_CHUNKS, N_CHUNKS)], isem))
    for c in idx_copies:
        c.wait()

    gather_copies = [None] * N_JOBS
    store_copies = [None] * N_SJ

    def fire_gathers(sj):
        b = sj % NB
        for g in range(GPS):
            i = sj * GPS + g
            tab, out, off = jobs[i]
            gather_copies[i] = pltpu.async_copy(
                tab.at[idx_v.at[i]], rows[b].at[pl.ds(g * CHUNK, CHUNK)],
                gsems[b])

    def fire_store(sj):
        b = sj % NB
        tab, out, off = jobs[sj * GPS]
        for g in range(GPS):
            gather_copies[sj * GPS + g].wait()
        store_copies[sj] = pltpu.async_copy(
            rows[b], out.at[pl.ds(off, GPS * CHUNK)], ssems[b])

    for sj in range(NB):
        fire_gathers(sj)

    stage_in.wait()
    pltpu.async_copy(
        stage_v, rel_sh.at[pl.ds(sid * REL_PER_TILE, REL_PER_TILE)],
        stsem).wait()

    for sj in range(NB):
        fire_store(sj)
    store_copies[0].wait()
    fire_gathers(3)

    plsc.subcore_barrier()

    for sj in range(NB + 1, N_SJ):
        store_copies[sj - NB].wait()
        fire_gathers(sj)
    for sj in range(NB, N_SJ):
        fire_store(sj)
    for sj in range(N_SJ - NB, N_SJ):
        store_copies[sj].wait()


@jax.jit
def _gather3(embedding, kgg_table, rel_table, scg_ids, relation_ids, kgg_ids):
    mesh = plsc.VectorSubcoreMesh(core_axis_name="c", subcore_axis_name="s")
    f = functools.partial(
        pl.kernel,
        mesh=mesh,
        out_type=(
            jax.ShapeDtypeStruct((B, D), jnp.float32),
            jax.ShapeDtypeStruct((B, D), jnp.float32),
            jax.ShapeDtypeStruct((B, D), jnp.float32),
        ),
        scratch_types=(
            [pltpu.VMEM((N_JOBS, CHUNK), jnp.int32),
             pltpu.VMEM((REL_PER_TILE, D), jnp.float32),
             pltpu.VMEM_SHARED((REL_PAD, D), jnp.float32)]
            + [pltpu.VMEM((GPS * CHUNK, D), jnp.float32) for _ in range(NB)]
            + [pltpu.SemaphoreType.DMA for _ in range(2 * NB + 2)]
        ),
    )(_gather3_body)
    rel_padded = jnp.concatenate(
        [rel_table,
         jnp.zeros((REL_PAD - rel_table.shape[0], D), rel_table.dtype)])
    return f(embedding, kgg_table, rel_padded,
             scg_ids.reshape(B // CHUNK, CHUNK),
             relation_ids.reshape(B // CHUNK, CHUNK),
             kgg_ids.reshape(B // CHUNK, CHUNK))


def kernel(embedding, kgg_table, rel_table, scg_ids, relation_ids, kgg_ids):
    scg_ids = scg_ids.astype(jnp.int32)
    relation_ids = relation_ids.astype(jnp.int32)
    kgg_ids = kgg_ids.astype(jnp.int32)
    return _gather3(embedding, kgg_table, rel_table,
                    scg_ids, relation_ids, kgg_ids)


# speedup vs baseline: 3.0233x; 1.0046x over previous
import functools

import jax
import jax.numpy as jnp
from jax import lax
from jax.experimental import pallas as pl
from jax.experimental.pallas import tpu as pltpu
from jax.experimental.pallas import tpu_sc as plsc

B = 16384
D = 128
NC = 2
NS = 16
NW = NC * NS
B_PER_W = B // NW
CHUNK = 128
N_CHUNKS = B_PER_W // CHUNK
N_JOBS = 3 * N_CHUNKS
GPS = 2
N_SJ = N_JOBS // GPS
NB = 3
N_REL = 1000
REL_PER_TILE = 64


def _gather3_body(emb_hbm, kgg_hbm, rel_hbm, scg_i_hbm, rel_i_hbm,
                  kgg_i_hbm, scg_out, kgg_out, rel_out,
                  idx_v, stage_v, rel_sh, *rest):
    rows = rest[:NB]
    isem = rest[NB]
    stsem = rest[NB + 1]
    gsems = rest[NB + 2:2 * NB + 2]
    ssems = rest[2 * NB + 2:]
    cid = lax.axis_index("c")
    sid = lax.axis_index("s")
    wid = sid * NC + cid
    base = wid * B_PER_W
    tables = (
        (emb_hbm, scg_i_hbm, scg_out),
        (kgg_hbm, kgg_i_hbm, kgg_out),
        (rel_sh, rel_i_hbm, rel_out),
    )
    jobs = []
    for tab, ids2d, out in tables:
        for j in range(N_CHUNKS):
            jobs.append((tab, out, base + j * CHUNK))

    stage_off = jnp.minimum(sid * REL_PER_TILE, N_REL - REL_PER_TILE)
    stage_in = pltpu.async_copy(
        rel_hbm.at[pl.ds(stage_off, REL_PER_TILE)], stage_v, stsem)
    idx_copies = []
    for t, (tab, ids2d, out) in enumerate(tables):
        idx_copies.append(pltpu.async_copy(
            ids2d.at[pl.ds(wid * N_CHUNKS, N_CHUNKS)],
            idx_v.at[pl.ds(t * N_CHUNKS, N_CHUNKS)], isem))
    for c in idx_copies:
        c.wait()

    gather_copies = [None] * N_JOBS
    store_copies = [None] * N_SJ

    def fire_gathers(sj):
        b = sj % NB
        for g in range(GPS):
            i = sj * GPS + g
            tab, out, off = jobs[i]
            gather_copies[i] = pltpu.async_copy(
                tab.at[idx_v.at[i]], rows[b].at[pl.ds(g * CHUNK, CHUNK)],
                gsems[b])

    def fire_store(sj):
        b = sj % NB
        tab, out, off = jobs[sj * GPS]
        for g in range(GPS):
            gather_copies[sj * GPS + g].wait()
        store_copies[sj] = pltpu.async_copy(
            rows[b], out.at[pl.ds(off, GPS * CHUNK)], ssems[b])

    for sj in range(NB):
        fire_gathers(sj)

    stage_in.wait()
    pltpu.async_copy(
        stage_v, rel_sh.at[pl.ds(stage_off, REL_PER_TILE)],
        stsem).wait()

    for sj in range(NB):
        fire_store(sj)
    store_copies[0].wait()
    fire_gathers(3)

    plsc.subcore_barrier()

    for sj in range(NB + 1, N_SJ):
        store_copies[sj - NB].wait()
        fire_gathers(sj)
    for sj in range(NB, N_SJ):
        fire_store(sj)
    for sj in range(N_SJ - NB, N_SJ):
        store_copies[sj].wait()


@jax.jit
def _gather3(embedding, kgg_table, rel_table, scg_ids, relation_ids, kgg_ids):
    mesh = plsc.VectorSubcoreMesh(core_axis_name="c", subcore_axis_name="s")
    f = functools.partial(
        pl.kernel,
        mesh=mesh,
        out_type=(
            jax.ShapeDtypeStruct((B, D), jnp.float32),
            jax.ShapeDtypeStruct((B, D), jnp.float32),
            jax.ShapeDtypeStruct((B, D), jnp.float32),
        ),
        scratch_types=(
            [pltpu.VMEM((N_JOBS, CHUNK), jnp.int32),
             pltpu.VMEM((REL_PER_TILE, D), jnp.float32),
             pltpu.VMEM_SHARED((N_REL, D), jnp.float32)]
            + [pltpu.VMEM((GPS * CHUNK, D), jnp.float32) for _ in range(NB)]
            + [pltpu.SemaphoreType.DMA for _ in range(2 * NB + 2)]
        ),
    )(_gather3_body)
    return f(embedding, kgg_table, rel_table,
             scg_ids.reshape(B // CHUNK, CHUNK),
             relation_ids.reshape(B // CHUNK, CHUNK),
             kgg_ids.reshape(B // CHUNK, CHUNK))


def kernel(embedding, kgg_table, rel_table, scg_ids, relation_ids, kgg_ids):
    scg_ids = scg_ids.astype(jnp.int32)
    relation_ids = relation_ids.astype(jnp.int32)
    kgg_ids = kgg_ids.astype(jnp.int32)
    return _gather3(embedding, kgg_table, rel_table,
                    scg_ids, relation_ids, kgg_ids)
